# baseline (device time: 29720 ns/iter reference)
import jax
import jax.numpy as jnp
from jax import lax
from jax.experimental import pallas as pl
from jax.experimental.pallas import tpu as pltpu

NZ = 4
NY = 4


def kernel(x):
    _, m, n_full = x.shape
    n_out = n_full // NZ
    m_slice = m // 8

    def body(x_hbm, out_ref, xstage, zsend_ref, zrecv_ref,
             stage_sem, zsend_sems, zrecv_sems,
             xsend_sems, xrecv_sems,
             yp_send, yp_recv, ym_send, ym_recv):
        my_x = lax.axis_index("x")
        my_y = lax.axis_index("y")
        my_z = lax.axis_index("z")
        s = my_y * 2 + my_x
        row0 = s * m_slice

        def _cl(v):
            return jnp.clip(v, 0, NY - 1)

        stage_cp = pltpu.make_async_copy(
            x_hbm.at[0, pl.ds(row0, m_slice), :], xstage, stage_sem
        )
        stage_cp.start()

        barrier_sem = pltpu.get_barrier_semaphore()
        for j in range(1, NZ):
            pl.semaphore_signal(
                barrier_sem, inc=1,
                device_id=(my_x, my_y, (my_z + j) % NZ),
                device_id_type=pl.DeviceIdType.MESH,
            )
        for j in range(1, NY):
            pl.semaphore_signal(
                barrier_sem, inc=1,
                device_id=(my_x, (my_y + j) % NY, my_z),
                device_id_type=pl.DeviceIdType.MESH,
            )
        pl.semaphore_signal(
            barrier_sem, inc=1,
            device_id=(1 - my_x, my_y, my_z),
            device_id_type=pl.DeviceIdType.MESH,
        )
        pl.semaphore_wait(barrier_sem, NZ - 1 + NY - 1 + 1)
        stage_cp.wait()

        z_rdmas = []
        for j in range(NZ - 1, 0, -1):
            c = (my_z + j) % NZ
            zsend_ref[j - 1, :, :] = xstage[
                :, pl.ds(c * n_out, n_out)
            ].astype(jnp.bfloat16)
            rdma = pltpu.make_async_remote_copy(
                src_ref=zsend_ref.at[j - 1],
                dst_ref=zrecv_ref.at[my_z],
                send_sem=zsend_sems.at[j - 1],
                recv_sem=zrecv_sems.at[j - 1],
                device_id=(my_x, my_y, c),
                device_id_type=pl.DeviceIdType.MESH,
            )
            rdma.start()
            z_rdmas.append(rdma)

        zrecv_ref[my_z, :, :] = xstage[
            :, pl.ds(my_z * n_out, n_out)
        ].astype(jnp.bfloat16)

        for j in range(1, NZ):
            recv = pltpu.make_async_remote_copy(
                src_ref=zsend_ref.at[j - 1],
                dst_ref=zrecv_ref.at[(my_z + NZ - j) % NZ],
                send_sem=zsend_sems.at[j - 1],
                recv_sem=zrecv_sems.at[j - 1],
                device_id=(my_x, my_y, my_z),
                device_id_type=pl.DeviceIdType.MESH,
            )
            recv.wait_recv()

        out_ref[pl.ds(row0, m_slice), :] = (
            zrecv_ref[0] + zrecv_ref[1] + zrecv_ref[2] + zrecv_ref[3]
        )

        y_sends = []
        for d in range(NY - 1, 0, -1):
            up = pltpu.make_async_remote_copy(
                src_ref=out_ref.at[pl.ds(row0, m_slice), :],
                dst_ref=out_ref.at[pl.ds(row0, m_slice), :],
                send_sem=yp_send.at[d - 1],
                recv_sem=yp_recv.at[d - 1],
                device_id=(my_x, _cl(my_y + d), my_z),
                device_id_type=pl.DeviceIdType.MESH,
            )
            dn = pltpu.make_async_remote_copy(
                src_ref=out_ref.at[pl.ds(row0, m_slice), :],
                dst_ref=out_ref.at[pl.ds(row0, m_slice), :],
                send_sem=ym_send.at[d - 1],
                recv_sem=ym_recv.at[d - 1],
                device_id=(my_x, _cl(my_y - d), my_z),
                device_id_type=pl.DeviceIdType.MESH,
            )

            @pl.when(my_y + d <= NY - 1)
            def _():
                up.start()

            @pl.when(my_y - d >= 0)
            def _():
                dn.start()

            y_sends.append((d, up, dn))

        x_own = pltpu.make_async_remote_copy(
            src_ref=out_ref.at[pl.ds(row0, m_slice), :],
            dst_ref=out_ref.at[pl.ds(row0, m_slice), :],
            send_sem=xsend_sems.at[0],
            recv_sem=xrecv_sems.at[0],
            device_id=(1 - my_x, my_y, my_z),
            device_id_type=pl.DeviceIdType.MESH,
        )
        x_own.start()

        x_fwds = []
        for d in range(1, NY):
            lo_row = (_cl(my_y - d) * 2 + my_x) * m_slice
            lo_recv = pltpu.make_async_remote_copy(
                src_ref=out_ref.at[pl.ds(row0, m_slice), :],
                dst_ref=out_ref.at[pl.ds(lo_row, m_slice), :],
                send_sem=yp_send.at[d - 1],
                recv_sem=yp_recv.at[d - 1],
                device_id=(my_x, my_y, my_z),
                device_id_type=pl.DeviceIdType.MESH,
            )
            lo_fwd = pltpu.make_async_remote_copy(
                src_ref=out_ref.at[pl.ds(lo_row, m_slice), :],
                dst_ref=out_ref.at[pl.ds(lo_row, m_slice), :],
                send_sem=xsend_sems.at[d],
                recv_sem=xrecv_sems.at[d],
                device_id=(1 - my_x, my_y, my_z),
                device_id_type=pl.DeviceIdType.MESH,
            )
            hi_row = (_cl(my_y + d) * 2 + my_x) * m_slice
            hi_recv = pltpu.make_async_remote_copy(
                src_ref=out_ref.at[pl.ds(row0, m_slice), :],
                dst_ref=out_ref.at[pl.ds(hi_row, m_slice), :],
                send_sem=ym_send.at[d - 1],
                recv_sem=ym_recv.at[d - 1],
                device_id=(my_x, my_y, my_z),
                device_id_type=pl.DeviceIdType.MESH,
            )
            hi_fwd = pltpu.make_async_remote_copy(
                src_ref=out_ref.at[pl.ds(hi_row, m_slice), :],
                dst_ref=out_ref.at[pl.ds(hi_row, m_slice), :],
                send_sem=xsend_sems.at[NY - 1 + d],
                recv_sem=xrecv_sems.at[NY - 1 + d],
                device_id=(1 - my_x, my_y, my_z),
                device_id_type=pl.DeviceIdType.MESH,
            )

            @pl.when(my_y - d >= 0)
            def _():
                lo_recv.wait_recv()
                lo_fwd.start()

            @pl.when(my_y + d <= NY - 1)
            def _():
                hi_recv.wait_recv()
                hi_fwd.start()

            x_fwds.append((d, lo_fwd, hi_fwd))

        x0_recv = pltpu.make_async_remote_copy(
            src_ref=out_ref.at[pl.ds(row0, m_slice), :],
            dst_ref=out_ref.at[pl.ds((my_y * 2 + 1 - my_x) * m_slice,
                                     m_slice), :],
            send_sem=xsend_sems.at[0],
            recv_sem=xrecv_sems.at[0],
            device_id=(my_x, my_y, my_z),
            device_id_type=pl.DeviceIdType.MESH,
        )
        x0_recv.wait_recv()
        for d in range(1, NY):
            lo_prow = (_cl(my_y - d) * 2 + 1 - my_x) * m_slice
            lo_precv = pltpu.make_async_remote_copy(
                src_ref=out_ref.at[pl.ds(row0, m_slice), :],
                dst_ref=out_ref.at[pl.ds(lo_prow, m_slice), :],
                send_sem=xsend_sems.at[d],
                recv_sem=xrecv_sems.at[d],
                device_id=(my_x, my_y, my_z),
                device_id_type=pl.DeviceIdType.MESH,
            )
            hi_prow = (_cl(my_y + d) * 2 + 1 - my_x) * m_slice
            hi_precv = pltpu.make_async_remote_copy(
                src_ref=out_ref.at[pl.ds(row0, m_slice), :],
                dst_ref=out_ref.at[pl.ds(hi_prow, m_slice), :],
                send_sem=xsend_sems.at[NY - 1 + d],
                recv_sem=xrecv_sems.at[NY - 1 + d],
                device_id=(my_x, my_y, my_z),
                device_id_type=pl.DeviceIdType.MESH,
            )

            @pl.when(my_y - d >= 0)
            def _():
                lo_precv.wait_recv()

            @pl.when(my_y + d <= NY - 1)
            def _():
                hi_precv.wait_recv()

        for rdma in z_rdmas:
            rdma.wait_send()
        for d, up, dn in y_sends:
            @pl.when(my_y + d <= NY - 1)
            def _():
                up.wait_send()

            @pl.when(my_y - d >= 0)
            def _():
                dn.wait_send()
        x_own.wait_send()
        for d, lo_fwd, hi_fwd in x_fwds:
            @pl.when(my_y - d >= 0)
            def _():
                lo_fwd.wait_send()

            @pl.when(my_y + d <= NY - 1)
            def _():
                hi_fwd.wait_send()

    return pl.pallas_call(
        body,
        out_shape=jax.ShapeDtypeStruct((m, n_out), jnp.bfloat16),
        in_specs=[pl.BlockSpec(memory_space=pltpu.MemorySpace.HBM)],
        out_specs=pl.BlockSpec(memory_space=pltpu.VMEM),
        scratch_shapes=[
            pltpu.VMEM((m_slice, n_full), jnp.float32),
            pltpu.VMEM((NZ - 1, m_slice, n_out), jnp.bfloat16),
            pltpu.VMEM((NZ, m_slice, n_out), jnp.bfloat16),
            pltpu.SemaphoreType.DMA,
            pltpu.SemaphoreType.DMA((NZ - 1,)),
            pltpu.SemaphoreType.DMA((NZ - 1,)),
            pltpu.SemaphoreType.DMA((2 * NY - 1,)),
            pltpu.SemaphoreType.DMA((2 * NY - 1,)),
            pltpu.SemaphoreType.DMA((NY - 1,)),
            pltpu.SemaphoreType.DMA((NY - 1,)),
            pltpu.SemaphoreType.DMA((NY - 1,)),
            pltpu.SemaphoreType.DMA((NY - 1,)),
        ],
        compiler_params=pltpu.CompilerParams(collective_id=0),
    )(x)


# device time: 25003 ns/iter; 1.1887x vs baseline; 1.1887x over previous
import jax
import jax.numpy as jnp
from jax import lax
from jax.experimental import pallas as pl
from jax.experimental.pallas import tpu as pltpu

NZ = 4
NY = 4
H = 2


def kernel(x):
    _, m, n_full = x.shape
    n_out = n_full // NZ
    m_slice = m // 8
    m_half = m_slice // H

    def body(x_hbm, out_ref, xstage, zsend_ref, zrecv_ref,
             stage_sem, zsend_sems, zrecv_sems,
             xsend_sems, xrecv_sems, ysend_sems, yrecv_sems):
        my_x = lax.axis_index("x")
        my_y = lax.axis_index("y")
        my_z = lax.axis_index("z")
        s = my_y * 2 + my_x
        row0 = s * m_slice

        stage_cp = pltpu.make_async_copy(
            x_hbm.at[0, pl.ds(row0, m_slice), :], xstage, stage_sem
        )
        stage_cp.start()

        barrier_sem = pltpu.get_barrier_semaphore()
        for j in range(1, NZ):
            pl.semaphore_signal(
                barrier_sem, inc=1,
                device_id=(my_x, my_y, (my_z + j) % NZ),
                device_id_type=pl.DeviceIdType.MESH,
            )
        for j in range(1, NY):
            pl.semaphore_signal(
                barrier_sem, inc=1,
                device_id=(my_x, (my_y + j) % NY, my_z),
                device_id_type=pl.DeviceIdType.MESH,
            )
        pl.semaphore_signal(
            barrier_sem, inc=1,
            device_id=(1 - my_x, my_y, my_z),
            device_id_type=pl.DeviceIdType.MESH,
        )
        pl.semaphore_wait(barrier_sem, NZ - 1 + NY - 1 + 1)
        stage_cp.wait()

        all_rdmas = []

        for rh in range(H):
            r0 = rh * m_half
            for j in range(1, NZ):
                c = (my_z + j) % NZ
                zsend_ref[rh, j - 1, :, :] = xstage[
                    pl.ds(r0, m_half), pl.ds(c * n_out, n_out)
                ].astype(jnp.bfloat16)
                rdma = pltpu.make_async_remote_copy(
                    src_ref=zsend_ref.at[rh, j - 1],
                    dst_ref=zrecv_ref.at[rh, my_z],
                    send_sem=zsend_sems.at[rh, j - 1],
                    recv_sem=zrecv_sems.at[rh, j - 1],
                    device_id=(my_x, my_y, c),
                    device_id_type=pl.DeviceIdType.MESH,
                )
                rdma.start()
                all_rdmas.append(rdma)
            zrecv_ref[rh, my_z, :, :] = xstage[
                pl.ds(r0, m_half), pl.ds(my_z * n_out, n_out)
            ].astype(jnp.bfloat16)

        def z_wait_reduce(rh):
            for j in range(1, NZ):
                recv = pltpu.make_async_remote_copy(
                    src_ref=zsend_ref.at[rh, j - 1],
                    dst_ref=zrecv_ref.at[rh, (my_z + NZ - j) % NZ],
                    send_sem=zsend_sems.at[rh, j - 1],
                    recv_sem=zrecv_sems.at[rh, j - 1],
                    device_id=(my_x, my_y, my_z),
                    device_id_type=pl.DeviceIdType.MESH,
                )
                recv.wait_recv()
            out_ref[pl.ds(row0 + rh * m_half, m_half), :] = (
                zrecv_ref[rh, 0] + zrecv_ref[rh, 1]
                + zrecv_ref[rh, 2] + zrecv_ref[rh, 3]
            )

        def gather_send(rh):
            src = out_ref.at[pl.ds(row0 + rh * m_half, m_half), :]
            for j in range(1, NY):
                ty = (my_y + j) % NY
                rdma = pltpu.make_async_remote_copy(
                    src_ref=src,
                    dst_ref=src,
                    send_sem=ysend_sems.at[rh, j - 1],
                    recv_sem=yrecv_sems.at[rh, j - 1],
                    device_id=(my_x, ty, my_z),
                    device_id_type=pl.DeviceIdType.MESH,
                )
                rdma.start()
                all_rdmas.append(rdma)
            x0 = pltpu.make_async_remote_copy(
                src_ref=src,
                dst_ref=src,
                send_sem=xsend_sems.at[rh, 0],
                recv_sem=xrecv_sems.at[rh, 0],
                device_id=(1 - my_x, my_y, my_z),
                device_id_type=pl.DeviceIdType.MESH,
            )
            x0.start()
            all_rdmas.append(x0)

        z_wait_reduce(0)
        gather_send(0)
        z_wait_reduce(1)
        gather_send(1)

        for rh in range(H):
            for j in range(1, NY):
                oy = (my_y + NY - j) % NY
                orow = (oy * 2 + my_x) * m_slice + rh * m_half
                recv = pltpu.make_async_remote_copy(
                    src_ref=out_ref.at[pl.ds(orow, m_half), :],
                    dst_ref=out_ref.at[pl.ds(orow, m_half), :],
                    send_sem=ysend_sems.at[rh, j - 1],
                    recv_sem=yrecv_sems.at[rh, j - 1],
                    device_id=(my_x, my_y, my_z),
                    device_id_type=pl.DeviceIdType.MESH,
                )
                recv.wait_recv()
                fwd = pltpu.make_async_remote_copy(
                    src_ref=out_ref.at[pl.ds(orow, m_half), :],
                    dst_ref=out_ref.at[pl.ds(orow, m_half), :],
                    send_sem=xsend_sems.at[rh, j],
                    recv_sem=xrecv_sems.at[rh, j],
                    device_id=(1 - my_x, my_y, my_z),
                    device_id_type=pl.DeviceIdType.MESH,
                )
                fwd.start()
                all_rdmas.append(fwd)

        for rh in range(H):
            for k in range(NY):
                oy = (my_y + NY - k) % NY
                prow = (oy * 2 + 1 - my_x) * m_slice + rh * m_half
                recv = pltpu.make_async_remote_copy(
                    src_ref=out_ref.at[pl.ds(prow, m_half), :],
                    dst_ref=out_ref.at[pl.ds(prow, m_half), :],
                    send_sem=xsend_sems.at[rh, k],
                    recv_sem=xrecv_sems.at[rh, k],
                    device_id=(my_x, my_y, my_z),
                    device_id_type=pl.DeviceIdType.MESH,
                )
                recv.wait_recv()

        for rdma in all_rdmas:
            rdma.wait_send()

    return pl.pallas_call(
        body,
        out_shape=jax.ShapeDtypeStruct((m, n_out), jnp.bfloat16),
        in_specs=[pl.BlockSpec(memory_space=pltpu.MemorySpace.HBM)],
        out_specs=pl.BlockSpec(memory_space=pltpu.VMEM),
        scratch_shapes=[
            pltpu.VMEM((m_slice, n_full), jnp.float32),
            pltpu.VMEM((H, NZ - 1, m_half, n_out), jnp.bfloat16),
            pltpu.VMEM((H, NZ, m_half, n_out), jnp.bfloat16),
            pltpu.SemaphoreType.DMA,
            pltpu.SemaphoreType.DMA((H, NZ - 1)),
            pltpu.SemaphoreType.DMA((H, NZ - 1)),
            pltpu.SemaphoreType.DMA((H, NY)),
            pltpu.SemaphoreType.DMA((H, NY)),
            pltpu.SemaphoreType.DMA((H, NY - 1)),
            pltpu.SemaphoreType.DMA((H, NY - 1)),
        ],
        compiler_params=pltpu.CompilerParams(collective_id=0),
    )(x)
